# baseline (speedup 1.0000x reference)
import jax
import jax.numpy as jnp
from jax.experimental import pallas as pl

ROWS = 16384
T = 49
F0 = 48
BLK = 4096
GRID = ROWS // BLK


def _body(x_ref, o_ref):
    v = x_ref[0]
    s = jnp.sum(v[0:8], axis=0) + v[8] + v[9]
    o_ref[...] = jnp.where(s > 0, jnp.ones_like(s), jnp.zeros_like(s))


@jax.jit
def kernel(x):
    xt = jnp.transpose(x, (1, 2, 0))
    out = pl.pallas_call(
        _body,
        grid=(GRID,),
        in_specs=[
            pl.BlockSpec((1, 16, BLK), lambda i: (T, F0 // 16, i)),
        ],
        out_specs=pl.BlockSpec((BLK,), lambda i: (i,)),
        out_shape=jax.ShapeDtypeStruct((ROWS,), jnp.float32),
    )(xt)
    return out.reshape(ROWS, 1)


# speedup vs baseline: 1.6640x; 1.6640x over previous
import jax
import jax.numpy as jnp
from jax.experimental import pallas as pl
from jax.experimental.pallas import tpu as pltpu

ROWS = 16384
T = 49
F0 = 48
NCHUNK = 4
CBLK = ROWS // NCHUNK


def _body(x_hbm, o_ref, *scratch):
    bufs = scratch[:NCHUNK]
    sems = scratch[NCHUNK:]
    copies = []
    for n in range(NCHUNK):
        cp = pltpu.make_async_copy(
            x_hbm.at[T, pl.ds(F0, 16), pl.ds(n * CBLK, CBLK)],
            bufs[n],
            sems[n],
        )
        cp.start()
        copies.append(cp)
    for n in range(NCHUNK):
        copies[n].wait()
        v = bufs[n]
        s = jnp.sum(v[0:8], axis=0) + v[8] + v[9]
        o_ref[pl.ds(n * CBLK, CBLK)] = jnp.where(
            s > 0, jnp.ones_like(s), jnp.zeros_like(s)
        )


@jax.jit
def kernel(x):
    xt = jnp.transpose(x, (1, 2, 0))
    out = pl.pallas_call(
        _body,
        in_specs=[pl.BlockSpec(memory_space=pl.ANY)],
        out_specs=pl.BlockSpec((ROWS,), lambda: (0,)),
        out_shape=jax.ShapeDtypeStruct((ROWS,), jnp.float32),
        scratch_shapes=(
            [pltpu.VMEM((16, CBLK), jnp.float32) for _ in range(NCHUNK)]
            + [pltpu.SemaphoreType.DMA for _ in range(NCHUNK)]
        ),
    )(xt)
    return out.reshape(ROWS, 1)
